# Initial kernel scaffold; baseline (speedup 1.0000x reference)
import jax
import jax.numpy as jnp
from jax.experimental import pallas as pl


def kernel(x):
    raise NotImplementedError("write your pallas kernel here")


# speedup vs baseline: 1.8229x; 1.8229x over previous
import functools

import jax
import jax.numpy as jnp
from jax import lax
from jax.experimental import pallas as pl
from jax.experimental.pallas import tpu as pltpu
from jax.experimental.pallas import tpu_sc as plsc

R = 128
N = 32768
K = 32
L = 16
NC, NS = 2, 16
NW = NC * NS
ROWS_PER_W = R // NW
NCHUNK = N // L

_NEG = float("-inf")


def _xlane_max(v):
    iota = lax.iota(jnp.int32, L)
    for sh in (8, 4, 2, 1):
        perm = iota ^ sh
        v = jnp.maximum(v, v.at[perm].get(mode="promise_in_bounds"))
    return v


def _popcount(m):
    return plsc.all_reduce_population_count(m)[0]


def _topk_one_row(row_v, cand_v, out_v):
    ninf = jnp.full((L,), _NEG, dtype=jnp.float32)
    iota = lax.iota(jnp.int32, L)

    def pa(i, carry):
        m1, m2 = carry
        v = row_v[pl.ds(i * L, L)]
        lo = jnp.minimum(m1, v)
        return jnp.maximum(m1, v), jnp.maximum(m2, lo)

    _, m2 = lax.fori_loop(0, NCHUNK, pa, (ninf, ninf))
    tv = -_xlane_max(-m2)

    def pb(i, cnt):
        v = row_v[pl.ds(i * L, L)]
        m = v >= tv
        plsc.store_compressed(cand_v.at[pl.ds(cnt, L)], v, mask=m)
        return cnt + _popcount(m)

    cnt = lax.fori_loop(0, NCHUNK, pb, jnp.int32(0))
    cand_v[pl.ds(cnt, L)] = ninf
    nch = cnt // L + 1

    def c_cond(s):
        return s < K

    def c_body(outcnt):
        def fm(j, m):
            return jnp.maximum(m, cand_v[pl.ds(j * L, L)])

        mx = lax.fori_loop(0, nch, fm, ninf)
        mv = _xlane_max(mx)

        def fc(j, c):
            v = cand_v[pl.ds(j * L, L)]
            eq = v == mv
            cand_v[pl.ds(j * L, L)] = jnp.where(eq, ninf, v)
            return c + _popcount(eq)

        c = lax.fori_loop(0, nch, fc, jnp.int32(0))
        r = jnp.minimum(c, K - outcnt)
        w1 = jnp.minimum(r, L)
        plsc.store_compressed(out_v.at[pl.ds(outcnt, L)], mv, mask=iota < w1)

        @pl.when(r > L)
        def _():
            plsc.store_compressed(
                out_v.at[pl.ds(outcnt + L, L)], mv, mask=iota < (r - L))

        return outcnt + r

    lax.while_loop(c_cond, c_body, jnp.int32(0))


def kernel(x):
    mesh = plsc.VectorSubcoreMesh(
        core_axis_name="c", subcore_axis_name="s",
        num_cores=NC, num_subcores=NS)

    @functools.partial(
        pl.kernel,
        out_type=jax.ShapeDtypeStruct((R * K,), jnp.float32),
        mesh=mesh,
        scratch_types=[
            pltpu.VMEM((N,), jnp.float32),
            pltpu.VMEM((N + L,), jnp.float32),
            pltpu.VMEM((K + 2 * L,), jnp.float32),
        ],
        compiler_params=pltpu.CompilerParams(needs_layout_passes=False),
    )
    def run(x_hbm, out_hbm, row_v, cand_v, out_v):
        wid = lax.axis_index("s") * NC + lax.axis_index("c")
        for j in range(ROWS_PER_W):
            r = wid * ROWS_PER_W + j
            pltpu.sync_copy(x_hbm.at[r], row_v)
            _topk_one_row(row_v, cand_v, out_v)
            pltpu.sync_copy(out_v.at[pl.ds(0, K)], out_hbm.at[pl.ds(r * K, K)])

    return run(x).reshape(R, K)


# speedup vs baseline: 3.4749x; 1.9063x over previous
import functools

import jax
import jax.numpy as jnp
from jax import lax
from jax.experimental import pallas as pl
from jax.experimental.pallas import tpu as pltpu
from jax.experimental.pallas import tpu_sc as plsc

R = 128
N = 32768
K = 32
L = 16
NC, NS = 2, 16
NW = NC * NS
ROWS_PER_W = R // NW
NCHUNK = N // L
GRP = 8
NGRP = NCHUNK // GRP

_NEG = float("-inf")


def _xlane_max(v):
    iota = lax.iota(jnp.int32, L)
    for sh in (8, 4, 2, 1):
        v = jnp.maximum(v, v.at[iota ^ sh].get(mode="promise_in_bounds"))
    return v


def _popcount(m):
    return plsc.all_reduce_population_count(m)[0]


def _topk_one_row(row_v, cand_v, out_v, obase):
    ninf = jnp.full((L,), _NEG, dtype=jnp.float32)
    iota = lax.iota(jnp.int32, L)

    def grp(i, carry):
        cnt, tv, m1, m2 = carry
        base = i * (GRP * L)
        vs = [row_v[pl.ds(base + k * L, L)] for k in range(GRP)]
        t0 = [jnp.maximum(vs[2 * k], vs[2 * k + 1]) for k in range(GRP // 2)]
        t1 = [jnp.maximum(t0[0], t0[1]), jnp.maximum(t0[2], t0[3])]
        g = jnp.maximum(t1[0], t1[1])
        lo = jnp.minimum(m1, g)
        m1n = jnp.maximum(m1, g)
        m2n = jnp.maximum(m2, lo)
        trig = _popcount(g >= tv) > 0

        def hot(op):
            cnt, _ = op
            ntv = -_xlane_max(-m2n)
            for k in range(GRP):
                mk = vs[k] >= ntv
                plsc.store_compressed(cand_v.at[pl.ds(cnt, L)], vs[k], mask=mk)
                cnt = cnt + _popcount(mk)
            return cnt, ntv

        cnt, tv = lax.cond(trig, hot, lambda op: op, (cnt, tv))
        return cnt, tv, m1n, m2n

    cnt, _, _, m2 = lax.fori_loop(
        0, NGRP, grp, (jnp.int32(0), ninf, ninf, ninf))
    tvf = -_xlane_max(-m2)

    cand_v[pl.ds(cnt, L)] = ninf
    nch = cnt // L + 1

    def rf(j, w):
        v = cand_v[pl.ds(j * L, L)]
        m = v >= tvf
        plsc.store_compressed(cand_v.at[pl.ds(w, L)], v, mask=m)
        return w + _popcount(m)

    cnt2 = lax.fori_loop(0, nch, rf, jnp.int32(0))
    cand_v[pl.ds(cnt2, L)] = ninf
    nch2 = cnt2 // L + 1

    def c_cond(s):
        return s < K

    def c_body(outcnt):
        def fm(j, m):
            return jnp.maximum(m, cand_v[pl.ds(j * L, L)])

        mx = lax.fori_loop(0, nch2, fm, ninf)
        mv = _xlane_max(mx)

        def fc(j, c):
            v = cand_v[pl.ds(j * L, L)]
            eq = v == mv
            cand_v[pl.ds(j * L, L)] = jnp.where(eq, ninf, v)
            return c + _popcount(eq)

        c = lax.fori_loop(0, nch2, fc, jnp.int32(0))
        r = jnp.minimum(c, K - outcnt)
        w1 = jnp.minimum(r, L)
        plsc.store_compressed(
            out_v.at[pl.ds(obase + outcnt, L)], mv, mask=iota < w1)

        @pl.when(r > L)
        def _():
            plsc.store_compressed(
                out_v.at[pl.ds(obase + outcnt + L, L)], mv, mask=iota < (r - L))

        return outcnt + r

    lax.while_loop(c_cond, c_body, jnp.int32(0))


def kernel(x):
    mesh = plsc.VectorSubcoreMesh(
        core_axis_name="c", subcore_axis_name="s",
        num_cores=NC, num_subcores=NS)

    @functools.partial(
        pl.kernel,
        out_type=jax.ShapeDtypeStruct((R * K,), jnp.float32),
        mesh=mesh,
        scratch_types=[
            pltpu.VMEM((N,), jnp.float32),
            pltpu.VMEM((N,), jnp.float32),
            pltpu.VMEM((N + L,), jnp.float32),
            pltpu.VMEM((ROWS_PER_W * K + 2 * L,), jnp.float32),
            pltpu.SemaphoreType.DMA,
            pltpu.SemaphoreType.DMA,
        ],
        compiler_params=pltpu.CompilerParams(needs_layout_passes=False),
    )
    def run(x_hbm, out_hbm, row0_v, row1_v, cand_v, out_v, sem0, sem1):
        wid = lax.axis_index("s") * NC + lax.axis_index("c")
        r0 = wid * ROWS_PER_W
        bufs = [row0_v, row1_v]
        sems = [sem0, sem1]
        cp = pltpu.async_copy(x_hbm.at[r0], row0_v, sem0)
        for j in range(ROWS_PER_W):
            nxt = None
            if j + 1 < ROWS_PER_W:
                nxt = pltpu.async_copy(
                    x_hbm.at[r0 + j + 1], bufs[(j + 1) % 2], sems[(j + 1) % 2])
            cp.wait()
            _topk_one_row(bufs[j % 2], cand_v, out_v, j * K)
            cp = nxt
        pltpu.sync_copy(
            out_v.at[pl.ds(0, ROWS_PER_W * K)],
            out_hbm.at[pl.ds(r0 * K, ROWS_PER_W * K)])

    return run(x).reshape(R, K)


# speedup vs baseline: 8.3756x; 2.4103x over previous
import functools

import jax
import jax.numpy as jnp
from jax import lax
from jax.experimental import pallas as pl
from jax.experimental.pallas import tpu as pltpu
from jax.experimental.pallas import tpu_sc as plsc

R = 128
N = 32768
K = 32
L = 16
NC, NS = 2, 16
NW = NC * NS
ROWS_PER_W = R // NW
NCHUNK = N // L
SUP = 16
NSUP = NCHUNK // SUP
NB = NSUP * L
P2U = 4

_NEG = float("-inf")


def _xlane_max(v):
    iota = lax.iota(jnp.int32, L)
    for sh in (8, 4, 2, 1):
        v = jnp.maximum(v, v.at[iota ^ sh].get(mode="promise_in_bounds"))
    return v


def _popcount(m):
    return plsc.all_reduce_population_count(m)[0]


def _sortd(v):
    s, _ = plsc.sort_key_val(v, v, descending=True)
    return s


def _topk_one_row(row_v, gmax_v, bidx_v, cand_v, out_v, obase):
    ninf = jnp.full((L,), _NEG, dtype=jnp.float32)
    iota = lax.iota(jnp.int32, L)

    def p1(s, carry):
        m1, m2 = carry
        base = s * (SUP * L)
        vs = [row_v[pl.ds(base + t * L, L)] for t in range(SUP)]
        while len(vs) > 1:
            vs = [jnp.maximum(vs[2 * i], vs[2 * i + 1])
                  for i in range(len(vs) // 2)]
        g = vs[0]
        gmax_v[pl.ds(s * L, L)] = g
        lo = jnp.minimum(m1, g)
        return jnp.maximum(m1, g), jnp.maximum(m2, lo)

    _, m2 = lax.fori_loop(0, NSUP, p1, (ninf, ninf))
    tvf = -_xlane_max(-m2)

    def p2(c, cntb):
        gvs = [gmax_v[pl.ds((c * P2U + u) * L, L)] for u in range(P2U)]
        ms = [gv >= tvf for gv in gvs]
        pcs = [_popcount(m) for m in ms]
        for u in range(P2U):
            idx = (c * P2U + u) * L + iota
            plsc.store_compressed(bidx_v.at[pl.ds(cntb, L)], idx, mask=ms[u])
            cntb = cntb + pcs[u]
        return cntb

    cntb = lax.fori_loop(0, NSUP // P2U, p2, jnp.int32(0))
    bidx_v[pl.ds(cntb, L)] = jnp.zeros((L,), jnp.int32)
    nbch = cntb // L + 1

    def p3(j, cnt):
        bv = bidx_v[pl.ds(j * L, L)]
        valid = (j * L + iota) < cntb
        sup = lax.shift_right_logical(bv, 4)
        lane = lax.bitwise_and(bv, 15)
        base = sup * (SUP * L) + lane
        vs, ms, pcs = [], [], []
        for t in range(SUP):
            v = plsc.load_gather(row_v, [base + t * L], mask=valid)
            m = jnp.logical_and(v >= tvf, valid)
            vs.append(v)
            ms.append(m)
            pcs.append(_popcount(m))
        for t in range(SUP):
            plsc.store_compressed(cand_v.at[pl.ds(cnt, L)], vs[t], mask=ms[t])
            cnt = cnt + pcs[t]
        return cnt

    cnt3 = lax.fori_loop(0, nbch, p3, jnp.int32(0))
    cand_v[pl.ds(cnt3, L)] = ninf
    nch3 = cnt3 // L + 1

    def fold(j, carry):
        hi, lo = carry
        s = _sortd(cand_v[pl.ds(j * L, L)])
        rs = jnp.flip(s)
        u = jnp.maximum(hi, rs)
        l = jnp.minimum(hi, rs)
        ls = _sortd(l)
        w = jnp.maximum(ls, jnp.flip(lo))
        return _sortd(u), _sortd(w)

    hi, lo = lax.fori_loop(0, nch3, fold, (ninf, ninf))
    out_v[pl.ds(obase, L)] = hi
    out_v[pl.ds(obase + L, L)] = lo


def kernel(x):
    mesh = plsc.VectorSubcoreMesh(
        core_axis_name="c", subcore_axis_name="s",
        num_cores=NC, num_subcores=NS)

    @functools.partial(
        pl.kernel,
        out_type=jax.ShapeDtypeStruct((R * K,), jnp.float32),
        mesh=mesh,
        scratch_types=[
            pltpu.VMEM((N,), jnp.float32),
            pltpu.VMEM((N,), jnp.float32),
            pltpu.VMEM((NB,), jnp.float32),
            pltpu.VMEM((NB + L,), jnp.int32),
            pltpu.VMEM((N + L,), jnp.float32),
            pltpu.VMEM((ROWS_PER_W * K + L,), jnp.float32),
            pltpu.SemaphoreType.DMA,
            pltpu.SemaphoreType.DMA,
        ],
        compiler_params=pltpu.CompilerParams(needs_layout_passes=False),
    )
    def run(x_hbm, out_hbm, row0_v, row1_v, gmax_v, bidx_v, cand_v, out_v,
            sem0, sem1):
        wid = lax.axis_index("s") * NC + lax.axis_index("c")
        r0 = wid * ROWS_PER_W
        bufs = [row0_v, row1_v]
        sems = [sem0, sem1]
        cp = pltpu.async_copy(x_hbm.at[r0], row0_v, sem0)
        for j in range(ROWS_PER_W):
            nxt = None
            if j + 1 < ROWS_PER_W:
                nxt = pltpu.async_copy(
                    x_hbm.at[r0 + j + 1], bufs[(j + 1) % 2], sems[(j + 1) % 2])
            cp.wait()
            _topk_one_row(bufs[j % 2], gmax_v, bidx_v, cand_v, out_v, j * K)
            cp = nxt
        pltpu.sync_copy(
            out_v.at[pl.ds(0, ROWS_PER_W * K)],
            out_hbm.at[pl.ds(r0 * K, ROWS_PER_W * K)])

    return run(x).reshape(R, K)


# speedup vs baseline: 8.6408x; 1.0317x over previous
import functools

import jax
import jax.numpy as jnp
from jax import lax
from jax.experimental import pallas as pl
from jax.experimental.pallas import tpu as pltpu
from jax.experimental.pallas import tpu_sc as plsc

R = 128
N = 32768
K = 32
L = 16
NC, NS = 2, 16
NW = NC * NS
ROWS_PER_W = R // NW
NCHUNK = N // L
SUP = 16
NSUP = NCHUNK // SUP
NB = NSUP * L
P2U = 8

_NEG = float("-inf")


def _xlane_max(v):
    iota = lax.iota(jnp.int32, L)
    for sh in (8, 4, 2, 1):
        v = jnp.maximum(v, v.at[iota ^ sh].get(mode="promise_in_bounds"))
    return v


def _popcount(m):
    return plsc.all_reduce_population_count(m)[0]


def _sortd(v):
    s, _ = plsc.sort_key_val(v, v, descending=True)
    return s


def _topk_one_row(row_v, gmax_v, bidx_v, cand_v, out_v, obase):
    ninf = jnp.full((L,), _NEG, dtype=jnp.float32)
    iota = lax.iota(jnp.int32, L)

    def p1(s, carry):
        m1, m2 = carry
        base = s * (SUP * L)
        g0 = row_v[pl.ds(base, L)]
        g1 = row_v[pl.ds(base + L, L)]
        g2 = row_v[pl.ds(base + 2 * L, L)]
        g3 = row_v[pl.ds(base + 3 * L, L)]
        for t in range(4, SUP, 4):
            g0 = jnp.maximum(g0, row_v[pl.ds(base + t * L, L)])
            g1 = jnp.maximum(g1, row_v[pl.ds(base + (t + 1) * L, L)])
            g2 = jnp.maximum(g2, row_v[pl.ds(base + (t + 2) * L, L)])
            g3 = jnp.maximum(g3, row_v[pl.ds(base + (t + 3) * L, L)])
        g = jnp.maximum(jnp.maximum(g0, g1), jnp.maximum(g2, g3))
        gmax_v[pl.ds(s * L, L)] = g
        lo = jnp.minimum(m1, g)
        return jnp.maximum(m1, g), jnp.maximum(m2, lo)

    _, m2 = lax.fori_loop(0, NSUP, p1, (ninf, ninf))
    tvf = -_xlane_max(-m2)

    def p2(c, cntb):
        gvs = [gmax_v[pl.ds((c * P2U + u) * L, L)] for u in range(P2U)]
        ms = [gv >= tvf for gv in gvs]
        pcs = [_popcount(m) for m in ms]
        for u in range(P2U):
            idx = (c * P2U + u) * L + iota
            plsc.store_compressed(bidx_v.at[pl.ds(cntb, L)], idx, mask=ms[u])
            cntb = cntb + pcs[u]
        return cntb

    cntb = lax.fori_loop(0, NSUP // P2U, p2, jnp.int32(0))
    bidx_v[pl.ds(cntb, L)] = jnp.zeros((L,), jnp.int32)
    nbch = cntb // L + 1

    def p3(j, cnt):
        bv = bidx_v[pl.ds(j * L, L)]
        valid = (j * L + iota) < cntb
        sup = lax.shift_right_logical(bv, 4)
        lane = lax.bitwise_and(bv, 15)
        base = sup * (SUP * L) + lane
        vs, ms, pcs = [], [], []
        for t in range(SUP):
            v = plsc.load_gather(row_v, [base + t * L], mask=valid)
            m = jnp.logical_and(v >= tvf, valid)
            vs.append(v)
            ms.append(m)
            pcs.append(_popcount(m))
        for t in range(SUP):
            plsc.store_compressed(cand_v.at[pl.ds(cnt, L)], vs[t], mask=ms[t])
            cnt = cnt + pcs[t]
        return cnt

    cnt3 = lax.fori_loop(0, nbch, p3, jnp.int32(0))
    cand_v[pl.ds(cnt3, L)] = ninf
    nch3 = cnt3 // L + 1

    def fold(j, carry):
        hi, lo = carry
        s = _sortd(cand_v[pl.ds(j * L, L)])
        rs = jnp.flip(s)
        u = jnp.maximum(hi, rs)
        l = jnp.minimum(hi, rs)
        ls = _sortd(l)
        w = jnp.maximum(ls, jnp.flip(lo))
        return _sortd(u), _sortd(w)

    hi, lo = lax.fori_loop(0, nch3, fold, (ninf, ninf))
    out_v[pl.ds(obase, L)] = hi
    out_v[pl.ds(obase + L, L)] = lo


def kernel(x):
    mesh = plsc.VectorSubcoreMesh(
        core_axis_name="c", subcore_axis_name="s",
        num_cores=NC, num_subcores=NS)

    @functools.partial(
        pl.kernel,
        out_type=jax.ShapeDtypeStruct((R * K,), jnp.float32),
        mesh=mesh,
        scratch_types=[
            pltpu.VMEM((N,), jnp.float32),
            pltpu.VMEM((N,), jnp.float32),
            pltpu.VMEM((NB,), jnp.float32),
            pltpu.VMEM((NB + L,), jnp.int32),
            pltpu.VMEM((N + L,), jnp.float32),
            pltpu.VMEM((ROWS_PER_W * K + L,), jnp.float32),
            pltpu.SemaphoreType.DMA,
            pltpu.SemaphoreType.DMA,
        ],
        compiler_params=pltpu.CompilerParams(needs_layout_passes=False),
    )
    def run(x_hbm, out_hbm, row0_v, row1_v, gmax_v, bidx_v, cand_v, out_v,
            sem0, sem1):
        wid = lax.axis_index("s") * NC + lax.axis_index("c")
        r0 = wid * ROWS_PER_W
        bufs = [row0_v, row1_v]
        sems = [sem0, sem1]
        cp = pltpu.async_copy(x_hbm.at[r0], row0_v, sem0)
        for j in range(ROWS_PER_W):
            nxt = None
            if j + 1 < ROWS_PER_W:
                nxt = pltpu.async_copy(
                    x_hbm.at[r0 + j + 1], bufs[(j + 1) % 2], sems[(j + 1) % 2])
            cp.wait()
            _topk_one_row(bufs[j % 2], gmax_v, bidx_v, cand_v, out_v, j * K)
            cp = nxt
        pltpu.sync_copy(
            out_v.at[pl.ds(0, ROWS_PER_W * K)],
            out_hbm.at[pl.ds(r0 * K, ROWS_PER_W * K)])

    return run(x).reshape(R, K)
